# baseline (speedup 1.0000x reference)
import functools
import numpy as np
import jax
import jax.numpy as jnp
from jax import lax
from jax.experimental import pallas as pl
from jax.experimental.pallas import tpu as pltpu
from jax.experimental.pallas import tpu_sc as plsc

_D = 64
_MAX_LEN = 512
_NUM_CORES = 2
_NUM_SUBCORES = 16
_NW = _NUM_CORES * _NUM_SUBCORES
_LANES = 16


def _make_pe2(max_len, d_model):
    position = np.arange(max_len, dtype=np.float32)[:, None]
    div_term = np.exp(
        np.arange(0, d_model, 2, dtype=np.float32) * -(np.log(10000.0) / d_model)
    )
    pe = np.zeros((max_len, d_model), dtype=np.float32)
    pe[:, 0::2] = np.sin(position * div_term)
    pe[:, 1::2] = np.cos(position * div_term)
    return pe * 2.0


@functools.partial(jax.jit, static_argnames=("batch", "seq_len"))
def _embed(seq_flat, pe2, table, batch, seq_len):
    chunks = []
    l0 = 0
    while l0 < seq_len:
        n = min(128, seq_len - l0)
        chunks.append((l0, n))
        l0 += n
    batches_per_w = batch // _NW

    mesh = plsc.VectorSubcoreMesh(
        core_axis_name="c", subcore_axis_name="s",
        num_cores=_NUM_CORES, num_subcores=_NUM_SUBCORES,
    )

    scratch = [pltpu.VMEM((seq_len, _D), jnp.float32)]
    for _, n in chunks:
        scratch.append(pltpu.VMEM((n,), jnp.int32))
        scratch.append(pltpu.VMEM((n, _D), jnp.float32))
    scratch.append(pltpu.SemaphoreType.DMA)

    @functools.partial(
        pl.kernel,
        out_type=jax.ShapeDtypeStruct((batch * seq_len, _D), jnp.float32),
        mesh=mesh,
        scratch_types=scratch,
        compiler_params=pltpu.CompilerParams(use_tc_tiling_on_sc=False),
    )
    def body(seq_hbm, pe2_hbm, table_hbm, out_hbm, pe2_v, *rest):
        bufs = []
        for i in range(len(chunks)):
            bufs.append((rest[2 * i], rest[2 * i + 1]))
        sem = rest[-1]
        wid = lax.axis_index("s") * _NUM_CORES + lax.axis_index("c")
        pltpu.sync_copy(pe2_hbm, pe2_v)

        def batch_body(bi, carry):
            gb = wid * batches_per_w + bi
            for (l0, n), (idx_v, row_v) in zip(chunks, bufs):
                row0 = gb * seq_len + l0
                pltpu.sync_copy(seq_hbm.at[pl.ds(row0, n)], idx_v)
                pltpu.async_copy(table_hbm.at[idx_v], row_v, sem).wait()

                def row_body(r, c2):
                    for j in range(_D // _LANES):
                        sl = pl.ds(j * _LANES, _LANES)
                        x = row_v[r, sl]
                        p = pe2_v[l0 + r, sl]
                        row_v[r, sl] = x + x + p
                    return c2

                lax.fori_loop(0, n, row_body, 0)
                pltpu.sync_copy(row_v, out_hbm.at[pl.ds(row0, n)])
            return carry

        lax.fori_loop(0, batches_per_w, batch_body, 0)

    return body(seq_flat, pe2, table)


def kernel(sequence, table):
    batch, seq_len = sequence.shape
    seq_flat = sequence.reshape(-1).astype(jnp.int32)
    pe2 = jnp.asarray(_make_pe2(_MAX_LEN, _D)[:seq_len])
    out = _embed(seq_flat, pe2, table, batch, seq_len)
    return out.reshape(batch, seq_len, _D)


# speedup vs baseline: 1.0006x; 1.0006x over previous
import functools
import numpy as np
import jax
import jax.numpy as jnp
from jax import lax
from jax.experimental import pallas as pl
from jax.experimental.pallas import tpu as pltpu
from jax.experimental.pallas import tpu_sc as plsc

_D = 64
_MAX_LEN = 512
_NUM_CORES = 2
_NUM_SUBCORES = 16
_NW = _NUM_CORES * _NUM_SUBCORES
_LANES = 16


def _make_pe2(max_len, d_model):
    position = np.arange(max_len, dtype=np.float32)[:, None]
    div_term = np.exp(
        np.arange(0, d_model, 2, dtype=np.float32) * -(np.log(10000.0) / d_model)
    )
    pe = np.zeros((max_len, d_model), dtype=np.float32)
    pe[:, 0::2] = np.sin(position * div_term)
    pe[:, 1::2] = np.cos(position * div_term)
    return pe * 2.0


@functools.partial(jax.jit, static_argnames=("batch", "seq_len"))
def _embed(seq, pe2, table, batch, seq_len):
    chunks = []
    l0 = 0
    while l0 < seq_len:
        n = min(128, seq_len - l0)
        chunks.append((l0, n))
        l0 += n
    batches_per_w = batch // _NW

    mesh = plsc.VectorSubcoreMesh(
        core_axis_name="c", subcore_axis_name="s",
        num_cores=_NUM_CORES, num_subcores=_NUM_SUBCORES,
    )

    scratch = [pltpu.VMEM((seq_len, _D), jnp.float32)]
    for _, n in chunks:
        scratch.append(pltpu.VMEM((n,), jnp.int32))
        scratch.append(pltpu.VMEM((n, _D), jnp.float32))
    scratch.append(pltpu.SemaphoreType.DMA)

    @functools.partial(
        pl.kernel,
        out_type=jax.ShapeDtypeStruct((batch, seq_len, _D), jnp.float32),
        mesh=mesh,
        scratch_types=scratch,
        compiler_params=pltpu.CompilerParams(use_tc_tiling_on_sc=False),
    )
    def body(seq_hbm, pe2_hbm, table_hbm, out_hbm, pe2_v, *rest):
        bufs = []
        for i in range(len(chunks)):
            bufs.append((rest[2 * i], rest[2 * i + 1]))
        sem = rest[-1]
        wid = lax.axis_index("s") * _NUM_CORES + lax.axis_index("c")
        pltpu.sync_copy(pe2_hbm, pe2_v)

        def batch_body(bi, carry):
            gb = wid * batches_per_w + bi
            for (l0, n), (idx_v, row_v) in zip(chunks, bufs):
                pltpu.sync_copy(seq_hbm.at[gb, pl.ds(l0, n)], idx_v)
                pltpu.async_copy(table_hbm.at[idx_v], row_v, sem).wait()

                def row_body(r, c2):
                    for j in range(_D // _LANES):
                        sl = pl.ds(j * _LANES, _LANES)
                        x = row_v[r, sl]
                        p = pe2_v[l0 + r, sl]
                        row_v[r, sl] = x + x + p
                    return c2

                lax.fori_loop(0, n, row_body, 0)
                pltpu.sync_copy(row_v, out_hbm.at[gb, pl.ds(l0, n)])
            return carry

        lax.fori_loop(0, batches_per_w, batch_body, 0)

    return body(seq, pe2, table)


def kernel(sequence, table):
    batch, seq_len = sequence.shape
    pe2 = jnp.asarray(_make_pe2(_MAX_LEN, _D)[:seq_len])
    return _embed(sequence.astype(jnp.int32), pe2, table, batch, seq_len)
